# baseline (speedup 1.0000x reference)
import functools

import jax
import jax.numpy as jnp
from jax import lax
from jax.experimental import pallas as pl
from jax.experimental.pallas import tpu as pltpu
from jax.experimental.pallas import tpu_sc as plsc

_N = 65536
_C = 1000
_R = 1024
_NW = 32
_CHUNK = _N // _NW
_BINS = 1024
_L = 16

_BN = 16384
_TCBLK0 = _BN // _R
_NBLK_TC = (_N - _BN) // _R
_RPT = _BN // _NW
_G = 32
_NG = _RPT // _G
_CP = 1008
_SCROWS = _BN // 2048


def _rows_body(x_ref, tgt_ref, out_ref):
    x = x_ref[...]
    tgt = tgt_ref[0, 0, :]
    m = jnp.max(x, axis=1, keepdims=True)
    col = lax.broadcasted_iota(jnp.int32, (_R, _C), 1)
    onehot = col == tgt[:, None]
    e = jnp.exp(x)
    sel = jnp.where(onehot, x, 0.0)
    s = jnp.sum(e, axis=1, keepdims=True)
    tlogit = jnp.sum(sel, axis=1, keepdims=True)
    ce = jnp.log(s) - tlogit
    signed = jnp.where(tlogit < m, ce, -ce)
    out_ref[0, 0, :] = signed[:, 0]


_rows_call = pl.pallas_call(
    _rows_body,
    grid=(_NBLK_TC,),
    in_specs=[
        pl.BlockSpec((_R, _C), lambda i: (i + _TCBLK0, 0)),
        pl.BlockSpec((1, 1, _R), lambda i: (i + _TCBLK0, 0, 0)),
    ],
    out_specs=pl.BlockSpec((1, 1, _R), lambda i: (i, 0, 0)),
    out_shape=jax.ShapeDtypeStruct((_NBLK_TC, 1, _R), jnp.float32),
)


@functools.partial(
    pl.kernel,
    out_type=[
        jax.ShapeDtypeStruct((_SCROWS, 2048), jnp.float32),
        jax.ShapeDtypeStruct((_SCROWS, 2048), jnp.float32),
    ],
    mesh=plsc.VectorSubcoreMesh(core_axis_name="c", subcore_axis_name="s"),
    compiler_params=pltpu.CompilerParams(needs_layout_passes=False,
                                         use_tc_tiling_on_sc=True),
    scratch_types=[
        pltpu.VMEM((_G, _C), jnp.float32),
        pltpu.VMEM((_G,), jnp.int32),
        pltpu.VMEM((_RPT,), jnp.int32),
        pltpu.VMEM((_RPT,), jnp.float32),
        pltpu.VMEM((_RPT,), jnp.float32),
        pltpu.VMEM((_G * _L,), jnp.float32),
        pltpu.VMEM((_G * _L,), jnp.float32),
        pltpu.SemaphoreType.DMA,
    ],
)
def _sc_dense(x_hbm, tgt_hbm, ssgn_hbm, tlog_hbm, rows_v, idx_v, tgt_v,
              s_v, t_v, accst, mxst, sem):
    wid = lax.axis_index("s") * 2 + lax.axis_index("c")
    row0 = wid * _RPT
    pltpu.sync_copy(tgt_hbm.at[pl.ds(row0, _RPT)], tgt_v)

    lane = lax.iota(jnp.int32, _L)
    tailmask = lane >= (_L - (_C % _L))

    def gbody(g, carry):
        gr0 = row0 + g * _G
        pltpu.async_copy(x_hbm.at[pl.ds(gr0, _G), :], rows_v, sem).wait()

        def rbody(r, carry2):
            acc = jnp.zeros((_L,), jnp.float32)
            mx = jnp.full((_L,), -1e30, jnp.float32)
            for k in range(_C // _L):
                v = rows_v[r, pl.ds(k * _L, _L)]
                acc = acc + jnp.exp(v)
                mx = jnp.maximum(mx, v)
            v = rows_v[r, pl.ds(_C - _L, _L)]
            acc = acc + jnp.where(tailmask, jnp.exp(v), 0.0)
            mx = jnp.maximum(mx, jnp.where(tailmask, v, -1e30))
            accst[pl.ds(r * _L, _L)] = acc
            mxst[pl.ds(r * _L, _L)] = mx
            return carry2

        lax.fori_loop(0, _G, rbody, 0)

        for sub in range(_G // _L):
            rloc16 = sub * _L + lane
            off = g * _G + sub * _L
            t16 = tgt_v[pl.ds(off, _L)]
            tvals = plsc.load_gather(rows_v, [rloc16, t16])
            s16 = jnp.zeros((_L,), jnp.float32)
            m16 = jnp.full((_L,), -1e30, jnp.float32)
            for l in range(_L):
                s16 = s16 + plsc.load_gather(accst, [rloc16 * _L + l])
                m16 = jnp.maximum(m16, plsc.load_gather(mxst, [rloc16 * _L + l]))
            s_v[pl.ds(off, _L)] = jnp.where(tvals < m16, s16, -s16)
            t_v[pl.ds(off, _L)] = tvals
        return carry

    lax.fori_loop(0, _NG, gbody, 0)

    pltpu.sync_copy(s_v, ssgn_hbm.at[wid // 4, pl.ds((wid % 4) * _RPT, _RPT)])
    pltpu.sync_copy(t_v, tlog_hbm.at[wid // 4, pl.ds((wid % 4) * _RPT, _RPT)])


def _sclog_body(s_ref, t_ref, out_ref):
    s = s_ref[...]
    t = t_ref[...]
    ce = jnp.log(jnp.abs(s)) - t
    out_ref[...] = jnp.where(s > 0, ce, -ce)


_sclog_call = pl.pallas_call(
    _sclog_body,
    out_shape=jax.ShapeDtypeStruct((_SCROWS, 2048), jnp.float32),
)


@functools.partial(
    pl.kernel,
    out_type=jax.ShapeDtypeStruct((_NW, 3 * _BINS), jnp.float32),
    mesh=plsc.VectorSubcoreMesh(core_axis_name="c", subcore_axis_name="s"),
    compiler_params=pltpu.CompilerParams(needs_layout_passes=False),
    scratch_types=[
        pltpu.VMEM((_CHUNK,), jnp.int32),
        pltpu.VMEM((_CHUNK,), jnp.float32),
        pltpu.VMEM((_BINS * _L,), jnp.float32),
        pltpu.VMEM((_BINS * _L,), jnp.float32),
        pltpu.VMEM((_BINS * _L,), jnp.float32),
        pltpu.VMEM((3 * _BINS,), jnp.float32),
    ],
)
def _hist_kernel(tgt_hbm, cesc_hbm, cetc_hbm, out_hbm,
                 tgt_v, cesgn_v, cnt_v, fn_v, ces_v, red_v):
    wid = lax.axis_index("s") * 2 + lax.axis_index("c")
    base = wid * _CHUNK
    pltpu.sync_copy(tgt_hbm.at[pl.ds(base, _CHUNK)], tgt_v)

    @pl.when(wid < _SCROWS)
    def _():
        pltpu.sync_copy(cesc_hbm.at[wid], cesgn_v)

    @pl.when(wid >= _SCROWS)
    def _():
        pltpu.sync_copy(
            cetc_hbm.at[pl.ds((wid - _SCROWS) * _CHUNK, _CHUNK)], cesgn_v)

    zero16 = jnp.zeros((_L,), jnp.float32)
    ones16 = jnp.ones((_L,), jnp.float32)
    lane = lax.iota(jnp.int32, _L)

    def zbody(r, carry):
        for k in range(4):
            sl = pl.ds((r * 4 + k) * _L, _L)
            cnt_v[sl] = zero16
            fn_v[sl] = zero16
            ces_v[sl] = zero16
        return carry

    lax.fori_loop(0, _BINS // 4, zbody, 0)

    def sbody(i, carry):
        for k in range(4):
            off = (i * 4 + k) * _L
            t16 = tgt_v[pl.ds(off, _L)] * _L + lane
            v16 = cesgn_v[pl.ds(off, _L)]
            idex16 = jnp.where(v16 > 0, 1.0, 0.0).astype(jnp.float32)
            plsc.addupdate_scatter(cnt_v, [t16], ones16)
            plsc.addupdate_scatter(fn_v, [t16], idex16)
            plsc.addupdate_scatter(ces_v, [t16], jnp.abs(v16))
        return carry

    lax.fori_loop(0, _CHUNK // (4 * _L), sbody, 0)

    def rbody(g, carry):
        b16 = (g * _L + lane) * _L
        for off, hist in ((0, cnt_v), (_BINS, fn_v), (2 * _BINS, ces_v)):
            tot = zero16
            for l in range(_L):
                tot = tot + plsc.load_gather(hist, [b16 + l])
            red_v[pl.ds(off + g * _L, _L)] = tot
        return carry

    lax.fori_loop(0, _BINS // _L, rbody, 0)

    pltpu.sync_copy(red_v, out_hbm.at[wid])


def _finish_body(p_ref, loss_ref):
    p = p_ref[...]
    s = jnp.sum(p, axis=0, keepdims=True)
    cnt = s[:, 0:_BINS]
    fn = s[:, _BINS:2 * _BINS]
    ces = s[:, 2 * _BINS:3 * _BINS]
    gt_counter = jnp.where(cnt > 0, cnt, 1.0)
    fn_counter = jnp.where(fn > 0, fn, 1.0)
    w = fn_counter / gt_counter
    loss_ref[...] = jnp.sum(w * ces, axis=1, keepdims=True) / jnp.float32(_N)


_finish_call = pl.pallas_call(
    _finish_body,
    out_shape=jax.ShapeDtypeStruct((1, 1), jnp.float32),
)


@jax.jit
def kernel(logits, target):
    tgt3 = target.reshape(_N // _R, 1, _R)
    ce_tc = _rows_call(logits, tgt3)
    ssgn, tlog = _sc_dense(logits, target)
    ce_sc = _sclog_call(ssgn, tlog)
    partials = _hist_kernel(target, ce_sc, ce_tc.reshape(_N - _BN))
    loss = _finish_call(partials)
    return loss[0, 0]


# speedup vs baseline: 1.0005x; 1.0005x over previous
import functools

import jax
import jax.numpy as jnp
from jax import lax
from jax.experimental import pallas as pl
from jax.experimental.pallas import tpu as pltpu
from jax.experimental.pallas import tpu_sc as plsc

_N = 65536
_C = 1000
_R = 1024
_NW = 32
_CHUNK = _N // _NW
_BINS = 1024
_L = 16

_BN = 16384
_TCBLK0 = _BN // _R
_NBLK_TC = (_N - _BN) // _R
_RPT = _BN // _NW
_G = 32
_NG = _RPT // _G
_CP = 1008
_SCROWS = _BN // 2048


def _rows_body(x_ref, tgt_ref, out_ref):
    x = x_ref[...]
    tgt = tgt_ref[0, 0, :]
    m = jnp.max(x, axis=1, keepdims=True)
    col = lax.broadcasted_iota(jnp.int32, (_R, _C), 1)
    onehot = col == tgt[:, None]
    e = jnp.exp(x)
    sel = jnp.where(onehot, x, 0.0)
    s = jnp.sum(e, axis=1, keepdims=True)
    tlogit = jnp.sum(sel, axis=1, keepdims=True)
    ce = jnp.log(s) - tlogit
    signed = jnp.where(tlogit < m, ce, -ce)
    out_ref[0, 0, :] = signed[:, 0]


_rows_call = pl.pallas_call(
    _rows_body,
    grid=(_NBLK_TC,),
    in_specs=[
        pl.BlockSpec((_R, _C), lambda i: (i + _TCBLK0, 0)),
        pl.BlockSpec((1, 1, _R), lambda i: (i + _TCBLK0, 0, 0)),
    ],
    out_specs=pl.BlockSpec((1, 1, _R), lambda i: (i, 0, 0)),
    out_shape=jax.ShapeDtypeStruct((_NBLK_TC, 1, _R), jnp.float32),
)


@functools.partial(
    pl.kernel,
    out_type=[
        jax.ShapeDtypeStruct((_SCROWS, 2048), jnp.float32),
        jax.ShapeDtypeStruct((_SCROWS, 2048), jnp.float32),
    ],
    mesh=plsc.VectorSubcoreMesh(core_axis_name="c", subcore_axis_name="s"),
    compiler_params=pltpu.CompilerParams(needs_layout_passes=False,
                                         use_tc_tiling_on_sc=True),
    scratch_types=[
        pltpu.VMEM((_G, _C), jnp.float32),
        pltpu.VMEM((_G,), jnp.int32),
        pltpu.VMEM((_RPT,), jnp.int32),
        pltpu.VMEM((_RPT,), jnp.float32),
        pltpu.VMEM((_RPT,), jnp.float32),
        pltpu.VMEM((_G * _L,), jnp.float32),
        pltpu.VMEM((_G * _L,), jnp.float32),
        pltpu.SemaphoreType.DMA,
    ],
)
def _sc_dense(x_hbm, tgt_hbm, ssgn_hbm, tlog_hbm, rows_v, idx_v, tgt_v,
              s_v, t_v, accst, mxst, sem):
    wid = lax.axis_index("s") * 2 + lax.axis_index("c")
    row0 = wid * _RPT
    pltpu.sync_copy(tgt_hbm.at[pl.ds(row0, _RPT)], tgt_v)

    lane = lax.iota(jnp.int32, _L)
    tailmask = lane >= (_L - (_C % _L))

    def gbody(g, carry):
        gr0 = row0 + g * _G
        pltpu.async_copy(x_hbm.at[pl.ds(gr0, _G), :], rows_v, sem).wait()

        def rbody(r, carry2):
            acc = jnp.zeros((_L,), jnp.float32)
            mx = jnp.full((_L,), -1e30, jnp.float32)
            for k in range(_C // _L):
                v = rows_v[r, pl.ds(k * _L, _L)]
                acc = acc + jnp.exp(v)
                mx = jnp.maximum(mx, v)
            v = rows_v[r, pl.ds(_C - _L, _L)]
            acc = acc + jnp.where(tailmask, jnp.exp(v), 0.0)
            mx = jnp.maximum(mx, jnp.where(tailmask, v, -1e30))
            accst[pl.ds(r * _L, _L)] = acc
            mxst[pl.ds(r * _L, _L)] = mx
            return carry2

        lax.fori_loop(0, _G, rbody, 0)

        for sub in range(_G // _L):
            rloc16 = sub * _L + lane
            off = g * _G + sub * _L
            t16 = tgt_v[pl.ds(off, _L)]
            tvals = plsc.load_gather(rows_v, [rloc16, t16])
            s16 = jnp.zeros((_L,), jnp.float32)
            m16 = jnp.full((_L,), -1e30, jnp.float32)
            for l in range(_L):
                s16 = s16 + plsc.load_gather(accst, [rloc16 * _L + l])
                m16 = jnp.maximum(m16, plsc.load_gather(mxst, [rloc16 * _L + l]))
            s_v[pl.ds(off, _L)] = jnp.where(tvals < m16, s16, -s16)
            t_v[pl.ds(off, _L)] = tvals
        return carry

    lax.fori_loop(0, _NG, gbody, 0)

    pltpu.sync_copy(s_v, ssgn_hbm.at[wid // 4, pl.ds((wid % 4) * _RPT, _RPT)])
    pltpu.sync_copy(t_v, tlog_hbm.at[wid // 4, pl.ds((wid % 4) * _RPT, _RPT)])


def _sclog_body(s_ref, t_ref, out_ref):
    s = s_ref[...]
    t = t_ref[...]
    ce = jnp.log(jnp.abs(s)) - t
    out_ref[...] = jnp.where(s > 0, ce, -ce)


_sclog_call = pl.pallas_call(
    _sclog_body,
    out_shape=jax.ShapeDtypeStruct((_SCROWS, 2048), jnp.float32),
)


@functools.partial(
    pl.kernel,
    out_type=jax.ShapeDtypeStruct((_NW, 3 * _BINS), jnp.float32),
    mesh=plsc.VectorSubcoreMesh(core_axis_name="c", subcore_axis_name="s"),
    compiler_params=pltpu.CompilerParams(needs_layout_passes=False),
    scratch_types=[
        pltpu.VMEM((_CHUNK,), jnp.int32),
        pltpu.VMEM((_CHUNK,), jnp.float32),
        pltpu.VMEM((_BINS * _L,), jnp.float32),
        pltpu.VMEM((_BINS * _L,), jnp.float32),
        pltpu.VMEM((_BINS * _L,), jnp.float32),
        pltpu.VMEM((3 * _BINS,), jnp.float32),
    ],
)
def _hist_kernel(tgt_hbm, cesc_hbm, cetc_hbm, out_hbm,
                 tgt_v, cesgn_v, cnt_v, fn_v, ces_v, red_v):
    wid = lax.axis_index("s") * 2 + lax.axis_index("c")
    base = wid * _CHUNK
    pltpu.sync_copy(tgt_hbm.at[pl.ds(base, _CHUNK)], tgt_v)

    @pl.when(wid < _SCROWS)
    def _():
        pltpu.sync_copy(cesc_hbm.at[wid], cesgn_v)

    @pl.when(wid >= _SCROWS)
    def _():
        pltpu.sync_copy(
            cetc_hbm.at[pl.ds((wid - _SCROWS) * _CHUNK, _CHUNK)], cesgn_v)

    zero16 = jnp.zeros((_L,), jnp.float32)
    ones16 = jnp.ones((_L,), jnp.float32)
    lane = lax.iota(jnp.int32, _L)

    def zbody(r, carry):
        for k in range(4):
            sl = pl.ds((r * 4 + k) * _L, _L)
            cnt_v[sl] = zero16
            fn_v[sl] = zero16
            ces_v[sl] = zero16
        return carry

    lax.fori_loop(0, _BINS // 4, zbody, 0)

    def sbody(i, carry):
        for k in range(4):
            off = (i * 4 + k) * _L
            t16 = tgt_v[pl.ds(off, _L)] * _L + lane
            v16 = cesgn_v[pl.ds(off, _L)]
            idex16 = jnp.where(v16 > 0, 1.0, 0.0).astype(jnp.float32)
            plsc.addupdate_scatter(cnt_v, [t16], ones16)
            plsc.addupdate_scatter(fn_v, [t16], idex16)
            plsc.addupdate_scatter(ces_v, [t16], jnp.abs(v16))
        return carry

    lax.fori_loop(0, _CHUNK // (4 * _L), sbody, 0)

    def rbody(g, carry):
        b16 = (g * _L + lane) * _L
        for off, hist in ((0, cnt_v), (_BINS, fn_v), (2 * _BINS, ces_v)):
            tot = zero16
            for l in range(_L):
                tot = tot + plsc.load_gather(hist, [b16 + l])
            red_v[pl.ds(off + g * _L, _L)] = tot
        return carry

    lax.fori_loop(0, _BINS // _L, rbody, 0)

    pltpu.sync_copy(red_v, out_hbm.at[wid])


def _finish_body(p_ref, loss_ref):
    p = p_ref[...]
    s = jnp.sum(p, axis=0, keepdims=True)
    cnt = s[:, 0:_BINS]
    fn = s[:, _BINS:2 * _BINS]
    ces = s[:, 2 * _BINS:3 * _BINS]
    gt_counter = jnp.where(cnt > 0, cnt, 1.0)
    fn_counter = jnp.where(fn > 0, fn, 1.0)
    w = fn_counter / gt_counter
    loss_ref[...] = jnp.sum(w * ces, axis=1, keepdims=True) / jnp.float32(_N)


_finish_call = pl.pallas_call(
    _finish_body,
    out_shape=jax.ShapeDtypeStruct((1, 1), jnp.float32),
)


@jax.jit
def kernel(logits, target):
    tgt3 = target.reshape(_N // _R, 1, _R)
    ssgn, tlog = _sc_dense(logits, target)
    ce_tc = _rows_call(logits, tgt3)
    ce_sc = _sclog_call(ssgn, tlog)
    partials = _hist_kernel(target, ce_sc, ce_tc.reshape(_N - _BN))
    loss = _finish_call(partials)
    return loss[0, 0]


# speedup vs baseline: 1.0421x; 1.0415x over previous
import functools

import jax
import jax.numpy as jnp
from jax import lax
from jax.experimental import pallas as pl
from jax.experimental.pallas import tpu as pltpu
from jax.experimental.pallas import tpu_sc as plsc

_N = 65536
_C = 1000
_R = 2048
_NW = 32
_CHUNK = _N // _NW
_BINS = 1024
_L = 16

_BN = 16384
_TCBLK0 = _BN // _R
_NBLK_TC = (_N - _BN) // _R
_RPT = _BN // _NW
_G = 32
_NG = _RPT // _G
_CP = 1008
_SCROWS = _BN // 2048


def _rows_body(x_ref, tgt_ref, out_ref):
    x = x_ref[...]
    tgt = tgt_ref[0, 0, :]
    m = jnp.max(x, axis=1, keepdims=True)
    col = lax.broadcasted_iota(jnp.int32, (_R, _C), 1)
    onehot = col == tgt[:, None]
    e = jnp.exp(x)
    sel = jnp.where(onehot, x, 0.0)
    s = jnp.sum(e, axis=1, keepdims=True)
    tlogit = jnp.sum(sel, axis=1, keepdims=True)
    ce = jnp.log(s) - tlogit
    signed = jnp.where(tlogit < m, ce, -ce)
    out_ref[0, 0, :] = signed[:, 0]


_rows_call = pl.pallas_call(
    _rows_body,
    grid=(_NBLK_TC,),
    in_specs=[
        pl.BlockSpec((_R, _C), lambda i: (i + _TCBLK0, 0)),
        pl.BlockSpec((1, 1, _R), lambda i: (i + _TCBLK0, 0, 0)),
    ],
    out_specs=pl.BlockSpec((1, 1, _R), lambda i: (i, 0, 0)),
    out_shape=jax.ShapeDtypeStruct((_NBLK_TC, 1, _R), jnp.float32),
)


@functools.partial(
    pl.kernel,
    out_type=[
        jax.ShapeDtypeStruct((_SCROWS, 2048), jnp.float32),
        jax.ShapeDtypeStruct((_SCROWS, 2048), jnp.float32),
    ],
    mesh=plsc.VectorSubcoreMesh(core_axis_name="c", subcore_axis_name="s"),
    compiler_params=pltpu.CompilerParams(needs_layout_passes=False,
                                         use_tc_tiling_on_sc=True),
    scratch_types=[
        pltpu.VMEM((_G, _C), jnp.float32),
        pltpu.VMEM((_G,), jnp.int32),
        pltpu.VMEM((_RPT,), jnp.int32),
        pltpu.VMEM((_RPT,), jnp.float32),
        pltpu.VMEM((_RPT,), jnp.float32),
        pltpu.VMEM((_G * _L,), jnp.float32),
        pltpu.VMEM((_G * _L,), jnp.float32),
        pltpu.SemaphoreType.DMA,
    ],
)
def _sc_dense(x_hbm, tgt_hbm, ssgn_hbm, tlog_hbm, rows_v, idx_v, tgt_v,
              s_v, t_v, accst, mxst, sem):
    wid = lax.axis_index("s") * 2 + lax.axis_index("c")
    row0 = wid * _RPT
    pltpu.sync_copy(tgt_hbm.at[pl.ds(row0, _RPT)], tgt_v)

    lane = lax.iota(jnp.int32, _L)
    tailmask = lane >= (_L - (_C % _L))

    def gbody(g, carry):
        gr0 = row0 + g * _G
        pltpu.async_copy(x_hbm.at[pl.ds(gr0, _G), :], rows_v, sem).wait()

        def rbody(r, carry2):
            acc = jnp.zeros((_L,), jnp.float32)
            mx = jnp.full((_L,), -1e30, jnp.float32)
            for k in range(_C // _L):
                v = rows_v[r, pl.ds(k * _L, _L)]
                acc = acc + jnp.exp(v)
                mx = jnp.maximum(mx, v)
            v = rows_v[r, pl.ds(_C - _L, _L)]
            acc = acc + jnp.where(tailmask, jnp.exp(v), 0.0)
            mx = jnp.maximum(mx, jnp.where(tailmask, v, -1e30))
            accst[pl.ds(r * _L, _L)] = acc
            mxst[pl.ds(r * _L, _L)] = mx
            return carry2

        lax.fori_loop(0, _G, rbody, 0)

        for sub in range(_G // _L):
            rloc16 = sub * _L + lane
            off = g * _G + sub * _L
            t16 = tgt_v[pl.ds(off, _L)]
            tvals = plsc.load_gather(rows_v, [rloc16, t16])
            s16 = jnp.zeros((_L,), jnp.float32)
            m16 = jnp.full((_L,), -1e30, jnp.float32)
            for l in range(_L):
                s16 = s16 + plsc.load_gather(accst, [rloc16 * _L + l])
                m16 = jnp.maximum(m16, plsc.load_gather(mxst, [rloc16 * _L + l]))
            s_v[pl.ds(off, _L)] = jnp.where(tvals < m16, s16, -s16)
            t_v[pl.ds(off, _L)] = tvals
        return carry

    lax.fori_loop(0, _NG, gbody, 0)

    pltpu.sync_copy(s_v, ssgn_hbm.at[wid // 4, pl.ds((wid % 4) * _RPT, _RPT)])
    pltpu.sync_copy(t_v, tlog_hbm.at[wid // 4, pl.ds((wid % 4) * _RPT, _RPT)])


def _sclog_body(s_ref, t_ref, out_ref):
    s = s_ref[...]
    t = t_ref[...]
    ce = jnp.log(jnp.abs(s)) - t
    out_ref[...] = jnp.where(s > 0, ce, -ce)


_sclog_call = pl.pallas_call(
    _sclog_body,
    out_shape=jax.ShapeDtypeStruct((_SCROWS, 2048), jnp.float32),
)


@functools.partial(
    pl.kernel,
    out_type=jax.ShapeDtypeStruct((_NW, 3 * _BINS), jnp.float32),
    mesh=plsc.VectorSubcoreMesh(core_axis_name="c", subcore_axis_name="s"),
    compiler_params=pltpu.CompilerParams(needs_layout_passes=False),
    scratch_types=[
        pltpu.VMEM((_CHUNK,), jnp.int32),
        pltpu.VMEM((_CHUNK,), jnp.float32),
        pltpu.VMEM((_BINS * _L,), jnp.float32),
        pltpu.VMEM((_BINS * _L,), jnp.float32),
        pltpu.VMEM((_BINS * _L,), jnp.float32),
        pltpu.VMEM((3 * _BINS,), jnp.float32),
    ],
)
def _hist_kernel(tgt_hbm, cesc_hbm, cetc_hbm, out_hbm,
                 tgt_v, cesgn_v, cnt_v, fn_v, ces_v, red_v):
    wid = lax.axis_index("s") * 2 + lax.axis_index("c")
    base = wid * _CHUNK
    pltpu.sync_copy(tgt_hbm.at[pl.ds(base, _CHUNK)], tgt_v)

    @pl.when(wid < _SCROWS)
    def _():
        pltpu.sync_copy(cesc_hbm.at[wid], cesgn_v)

    @pl.when(wid >= _SCROWS)
    def _():
        pltpu.sync_copy(
            cetc_hbm.at[pl.ds((wid - _SCROWS) * _CHUNK, _CHUNK)], cesgn_v)

    zero16 = jnp.zeros((_L,), jnp.float32)
    ones16 = jnp.ones((_L,), jnp.float32)
    lane = lax.iota(jnp.int32, _L)

    def zbody(r, carry):
        for k in range(4):
            sl = pl.ds((r * 4 + k) * _L, _L)
            cnt_v[sl] = zero16
            fn_v[sl] = zero16
            ces_v[sl] = zero16
        return carry

    lax.fori_loop(0, _BINS // 4, zbody, 0)

    def sbody(i, carry):
        for k in range(4):
            off = (i * 4 + k) * _L
            t16 = tgt_v[pl.ds(off, _L)] * _L + lane
            v16 = cesgn_v[pl.ds(off, _L)]
            idex16 = jnp.where(v16 > 0, 1.0, 0.0).astype(jnp.float32)
            plsc.addupdate_scatter(cnt_v, [t16], ones16)
            plsc.addupdate_scatter(fn_v, [t16], idex16)
            plsc.addupdate_scatter(ces_v, [t16], jnp.abs(v16))
        return carry

    lax.fori_loop(0, _CHUNK // (4 * _L), sbody, 0)

    def rbody(g, carry):
        b16 = (g * _L + lane) * _L
        for off, hist in ((0, cnt_v), (_BINS, fn_v), (2 * _BINS, ces_v)):
            tot = zero16
            for l in range(_L):
                tot = tot + plsc.load_gather(hist, [b16 + l])
            red_v[pl.ds(off + g * _L, _L)] = tot
        return carry

    lax.fori_loop(0, _BINS // _L, rbody, 0)

    pltpu.sync_copy(red_v, out_hbm.at[wid])


def _finish_body(p_ref, loss_ref):
    p = p_ref[...]
    s = jnp.sum(p, axis=0, keepdims=True)
    cnt = s[:, 0:_BINS]
    fn = s[:, _BINS:2 * _BINS]
    ces = s[:, 2 * _BINS:3 * _BINS]
    gt_counter = jnp.where(cnt > 0, cnt, 1.0)
    fn_counter = jnp.where(fn > 0, fn, 1.0)
    w = fn_counter / gt_counter
    loss_ref[...] = jnp.sum(w * ces, axis=1, keepdims=True) / jnp.float32(_N)


_finish_call = pl.pallas_call(
    _finish_body,
    out_shape=jax.ShapeDtypeStruct((1, 1), jnp.float32),
)


@jax.jit
def kernel(logits, target):
    tgt3 = target.reshape(_N // _R, 1, _R)
    ssgn, tlog = _sc_dense(logits, target)
    ce_tc = _rows_call(logits, tgt3)
    ce_sc = _sclog_call(ssgn, tlog)
    partials = _hist_kernel(target, ce_sc, ce_tc.reshape(_N - _BN))
    loss = _finish_call(partials)
    return loss[0, 0]


# speedup vs baseline: 1.0435x; 1.0014x over previous
import functools

import jax
import jax.numpy as jnp
from jax import lax
from jax.experimental import pallas as pl
from jax.experimental.pallas import tpu as pltpu
from jax.experimental.pallas import tpu_sc as plsc

_N = 65536
_C = 1000
_R = 4096
_NW = 32
_CHUNK = _N // _NW
_BINS = 1024
_L = 16

_BN = 16384
_TCBLK0 = 0
_NBLK_TC = (_N - _BN) // _R
_RPT = _BN // _NW
_G = 32
_NG = _RPT // _G
_CP = 1008
_SCROWS = _BN // 2048


def _rows_body(x_ref, tgt_ref, out_ref):
    x = x_ref[...]
    tgt = tgt_ref[0, 0, :]
    m = jnp.max(x, axis=1, keepdims=True)
    col = lax.broadcasted_iota(jnp.int32, (_R, _C), 1)
    onehot = col == tgt[:, None]
    e = jnp.exp(x)
    sel = jnp.where(onehot, x, 0.0)
    s = jnp.sum(e, axis=1, keepdims=True)
    tlogit = jnp.sum(sel, axis=1, keepdims=True)
    ce = jnp.log(s) - tlogit
    signed = jnp.where(tlogit < m, ce, -ce)
    out_ref[0, 0, :] = signed[:, 0]


_rows_call = pl.pallas_call(
    _rows_body,
    grid=(_NBLK_TC,),
    in_specs=[
        pl.BlockSpec((_R, _C), lambda i: (i + _BN // _R, 0)),
        pl.BlockSpec((1, 1, _R), lambda i: (i + _TCBLK0, 0, 0)),
    ],
    out_specs=pl.BlockSpec((1, 1, _R), lambda i: (i, 0, 0)),
    out_shape=jax.ShapeDtypeStruct((_NBLK_TC, 1, _R), jnp.float32),
)


@functools.partial(
    pl.kernel,
    out_type=[
        jax.ShapeDtypeStruct((_SCROWS, 2048), jnp.float32),
        jax.ShapeDtypeStruct((_SCROWS, 2048), jnp.float32),
    ],
    mesh=plsc.VectorSubcoreMesh(core_axis_name="c", subcore_axis_name="s"),
    compiler_params=pltpu.CompilerParams(needs_layout_passes=False,
                                         use_tc_tiling_on_sc=True),
    scratch_types=[
        pltpu.VMEM((_G, _C), jnp.float32),
        pltpu.VMEM((_G,), jnp.int32),
        pltpu.VMEM((_RPT,), jnp.int32),
        pltpu.VMEM((_RPT,), jnp.float32),
        pltpu.VMEM((_RPT,), jnp.float32),
        pltpu.VMEM((_G * _L,), jnp.float32),
        pltpu.VMEM((_G * _L,), jnp.float32),
        pltpu.SemaphoreType.DMA,
    ],
)
def _sc_dense(x_hbm, tgt_hbm, ssgn_hbm, tlog_hbm, rows_v, idx_v, tgt_v,
              s_v, t_v, accst, mxst, sem):
    wid = lax.axis_index("s") * 2 + lax.axis_index("c")
    row0 = wid * _RPT
    pltpu.sync_copy(tgt_hbm.at[pl.ds(row0, _RPT)], tgt_v)

    lane = lax.iota(jnp.int32, _L)
    tailmask = lane >= (_L - (_C % _L))

    def gbody(g, carry):
        gr0 = row0 + g * _G
        pltpu.async_copy(x_hbm.at[pl.ds(gr0, _G), :], rows_v, sem).wait()

        def rbody(r, carry2):
            acc = jnp.zeros((_L,), jnp.float32)
            mx = jnp.full((_L,), -1e30, jnp.float32)
            for k in range(_C // _L):
                v = rows_v[r, pl.ds(k * _L, _L)]
                acc = acc + jnp.exp(v)
                mx = jnp.maximum(mx, v)
            v = rows_v[r, pl.ds(_C - _L, _L)]
            acc = acc + jnp.where(tailmask, jnp.exp(v), 0.0)
            mx = jnp.maximum(mx, jnp.where(tailmask, v, -1e30))
            accst[pl.ds(r * _L, _L)] = acc
            mxst[pl.ds(r * _L, _L)] = mx
            return carry2

        lax.fori_loop(0, _G, rbody, 0)

        for sub in range(_G // _L):
            rloc16 = sub * _L + lane
            off = g * _G + sub * _L
            t16 = tgt_v[pl.ds(off, _L)]
            tvals = plsc.load_gather(rows_v, [rloc16, t16])
            s16 = jnp.zeros((_L,), jnp.float32)
            m16 = jnp.full((_L,), -1e30, jnp.float32)
            for l in range(_L):
                s16 = s16 + plsc.load_gather(accst, [rloc16 * _L + l])
                m16 = jnp.maximum(m16, plsc.load_gather(mxst, [rloc16 * _L + l]))
            s_v[pl.ds(off, _L)] = jnp.where(tvals < m16, s16, -s16)
            t_v[pl.ds(off, _L)] = tvals
        return carry

    lax.fori_loop(0, _NG, gbody, 0)

    pltpu.sync_copy(s_v, ssgn_hbm.at[wid // 4, pl.ds((wid % 4) * _RPT, _RPT)])
    pltpu.sync_copy(t_v, tlog_hbm.at[wid // 4, pl.ds((wid % 4) * _RPT, _RPT)])


def _sclog_body(s_ref, t_ref, out_ref):
    s = s_ref[...]
    t = t_ref[...]
    ce = jnp.log(jnp.abs(s)) - t
    out_ref[...] = jnp.where(s > 0, ce, -ce)


_sclog_call = pl.pallas_call(
    _sclog_body,
    out_shape=jax.ShapeDtypeStruct((_SCROWS, 2048), jnp.float32),
)


@functools.partial(
    pl.kernel,
    out_type=jax.ShapeDtypeStruct((_NW, 3 * _BINS), jnp.float32),
    mesh=plsc.VectorSubcoreMesh(core_axis_name="c", subcore_axis_name="s"),
    compiler_params=pltpu.CompilerParams(needs_layout_passes=False),
    scratch_types=[
        pltpu.VMEM((_CHUNK,), jnp.int32),
        pltpu.VMEM((_CHUNK,), jnp.float32),
        pltpu.VMEM((_BINS * _L,), jnp.float32),
        pltpu.VMEM((_BINS * _L,), jnp.float32),
        pltpu.VMEM((_BINS * _L,), jnp.float32),
        pltpu.VMEM((3 * _BINS,), jnp.float32),
    ],
)
def _hist_kernel(tgt_hbm, cesc_hbm, cetc_hbm, out_hbm,
                 tgt_v, cesgn_v, cnt_v, fn_v, ces_v, red_v):
    wid = lax.axis_index("s") * 2 + lax.axis_index("c")
    base = wid * _CHUNK
    pltpu.sync_copy(tgt_hbm.at[pl.ds(base, _CHUNK)], tgt_v)

    @pl.when(wid < _SCROWS)
    def _():
        pltpu.sync_copy(cesc_hbm.at[wid], cesgn_v)

    @pl.when(wid >= _SCROWS)
    def _():
        pltpu.sync_copy(
            cetc_hbm.at[pl.ds((wid - _SCROWS) * _CHUNK, _CHUNK)], cesgn_v)

    zero16 = jnp.zeros((_L,), jnp.float32)
    ones16 = jnp.ones((_L,), jnp.float32)
    lane = lax.iota(jnp.int32, _L)

    def zbody(r, carry):
        for k in range(4):
            sl = pl.ds((r * 4 + k) * _L, _L)
            cnt_v[sl] = zero16
            fn_v[sl] = zero16
            ces_v[sl] = zero16
        return carry

    lax.fori_loop(0, _BINS // 4, zbody, 0)

    def sbody(i, carry):
        for k in range(4):
            off = (i * 4 + k) * _L
            t16 = tgt_v[pl.ds(off, _L)] * _L + lane
            v16 = cesgn_v[pl.ds(off, _L)]
            idex16 = jnp.where(v16 > 0, 1.0, 0.0).astype(jnp.float32)
            plsc.addupdate_scatter(cnt_v, [t16], ones16)
            plsc.addupdate_scatter(fn_v, [t16], idex16)
            plsc.addupdate_scatter(ces_v, [t16], jnp.abs(v16))
        return carry

    lax.fori_loop(0, _CHUNK // (4 * _L), sbody, 0)

    def rbody(g, carry):
        b16 = (g * _L + lane) * _L
        for off, hist in ((0, cnt_v), (_BINS, fn_v), (2 * _BINS, ces_v)):
            tot = zero16
            for l in range(_L):
                tot = tot + plsc.load_gather(hist, [b16 + l])
            red_v[pl.ds(off + g * _L, _L)] = tot
        return carry

    lax.fori_loop(0, _BINS // _L, rbody, 0)

    pltpu.sync_copy(red_v, out_hbm.at[wid])


def _finish_body(p_ref, loss_ref):
    p = p_ref[...]
    s = jnp.sum(p, axis=0, keepdims=True)
    cnt = s[:, 0:_BINS]
    fn = s[:, _BINS:2 * _BINS]
    ces = s[:, 2 * _BINS:3 * _BINS]
    gt_counter = jnp.where(cnt > 0, cnt, 1.0)
    fn_counter = jnp.where(fn > 0, fn, 1.0)
    w = fn_counter / gt_counter
    loss_ref[...] = jnp.sum(w * ces, axis=1, keepdims=True) / jnp.float32(_N)


_finish_call = pl.pallas_call(
    _finish_body,
    out_shape=jax.ShapeDtypeStruct((1, 1), jnp.float32),
)


@jax.jit
def kernel(logits, target):
    tgt3 = target.reshape(_N // _R, 1, _R)
    ssgn, tlog = _sc_dense(logits, target)
    ce_tc = _rows_call(logits, tgt3)
    ce_sc = _sclog_call(ssgn, tlog)
    partials = _hist_kernel(target, ce_sc, ce_tc.reshape(_N - _BN))
    loss = _finish_call(partials)
    return loss[0, 0]
